# Initial kernel scaffold; baseline (speedup 1.0000x reference)
import jax
import jax.numpy as jnp
from jax.experimental import pallas as pl


def kernel(pred_s1, pred_s2, pred_s3, bboxes, labels, anchors):
    raise NotImplementedError("write your pallas kernel here")


# speedup vs baseline: 13.1443x; 13.1443x over previous
import jax
import jax.numpy as jnp
from jax.experimental import pallas as pl
from jax.experimental.pallas import tpu as pltpu

_C = 80
_V = 5 + _C


def _two_sum(a, b):
    s = a + b
    bv = s - a
    e = (a - (s - bv)) + (b - bv)
    return s, e


def _floor_scaled(xs, xe, gf):
    p = xs * gf
    c = jnp.float32(4097.0) * xs
    xhi = c - (c - xs)
    xlo = xs - xhi
    err = (xhi * gf - p) + xlo * gf
    lo = err + xe * gf
    c0 = jnp.floor(p)
    r = p - c0
    adj = jnp.where(lo >= (jnp.float32(1.0) - r), 1.0,
                    jnp.where(lo < -r, -1.0, 0.0))
    return (c0 + adj).astype(jnp.int32)


def _softplus(x):
    return jnp.maximum(x, 0.0) + jnp.log1p(jnp.exp(-jnp.abs(x)))


def _make_assign_body(B, N, grids):
    g1, g2, g3 = grids
    c1 = g1 * g1 * 3
    c2 = g2 * g2 * 3

    def body(bx_ref, by_ref, bw_ref, bh_ref, lab_ref, aw_ref, ah_ref,
             rows1_ref, rows2_ref, rows3_ref, base_ref, labl_ref, nobj_ref):
        x = bx_ref[...]
        w = bw_ref[...]
        y = by_ref[...]
        h = bh_ref[...]
        cxs, cxe = _two_sum(x, w * 0.5)
        cys, cye = _two_sum(y, h * 0.5)
        aw = aw_ref[...]
        ah = ah_ref[...]
        d1, e1 = _two_sum(w[:, :, None], -aw)
        d2, e2 = _two_sum(h[:, :, None], -ah)
        a1 = jnp.abs(d1)
        ae1 = jnp.where(d1 < 0, -e1, e1)
        a2 = jnp.abs(d2)
        ae2 = jnp.where(d2 < 0, -e2, e2)
        hi0, er = _two_sum(a1, a2)
        lo0 = er + ae1 + ae2
        t = hi0 + lo0
        lo = lo0 - (t - hi0)
        hi = t
        hm = jnp.min(hi, axis=-1, keepdims=True)
        m1 = hi == hm
        lm = jnp.min(jnp.where(m1, lo, jnp.float32(jnp.inf)), axis=-1,
                     keepdims=True)
        m2 = m1 & (lo == lm)
        j9 = jax.lax.broadcasted_iota(jnp.int32, m2.shape, 2)
        prior = jnp.min(jnp.where(m2, j9, 9), axis=-1)
        tidx = prior // 3
        aidx = prior - tidx * 3
        g = jnp.where(tidx == 0, g1, jnp.where(tidx == 1, g2, g3))
        gf = g.astype(jnp.float32)
        cxi = _floor_scaled(cxs, cxe, gf)
        cyi = _floor_scaled(cys, cye, gf)
        valid = (cyi >= 0) & (cyi < g) & (cxi >= 0) & (cxi < g)
        offs = jnp.where(tidx == 0, 0, jnp.where(tidx == 1, c1, c1 + c2))
        key = offs + (cyi * g + cxi) * 3 + aidx
        eq = key[:, :, None] == key[:, None, :]
        vm = valid[:, None, :]
        nn = jax.lax.broadcasted_iota(jnp.int32, (B, N, N), 1)
        mm = jax.lax.broadcasted_iota(jnp.int32, (B, N, N), 2)
        clob = jnp.sum((eq & vm & (mm > nn)).astype(jnp.int32), axis=2) > 0
        winner = valid & jnp.logical_not(clob)
        b2 = jax.lax.broadcasted_iota(jnp.int32, (B, N), 0)
        for s, gv, rref in ((0, g1, rows1_ref), (1, g2, rows2_ref),
                            (2, g3, rows3_ref)):
            sel = winner & (tidx == s)
            row = (b2 * gv + cyi) * gv + cxi
            rref[...] = jnp.where(sel, row, -1)
        base_ref[...] = aidx * _V
        labl_ref[...] = aidx * _V + 5 + lab_ref[...]
        nobj = jnp.sum(winner.astype(jnp.float32))
        ri = jax.lax.broadcasted_iota(jnp.int32, (8, 128), 0)
        ci = jax.lax.broadcasted_iota(jnp.int32, (8, 128), 1)
        nobj_ref[...] = jnp.where((ri == 0) & (ci == 0), nobj, 0.0)

    return body


def _gather_body(rows_ref, base_ref, labl_ref, x_ref, acc_ref):
    k = pl.program_id(0)
    row = rows_ref[k]
    wgt = jnp.where(row >= 0, jnp.float32(1.0), jnp.float32(0.0))
    b = base_ref[k]
    ll = labl_ref[k]
    x = x_ref[0]
    lanes = jax.lax.broadcasted_iota(jnp.int32, x.shape, 1)
    sp = _softplus(x)
    isconf = lanes == b + 4
    spc = jnp.sum(jnp.where(isconf, sp, 0.0))
    xc = jnp.sum(jnp.where(isconf, x, 0.0))
    clsm = (lanes >= b + 5) & (lanes < b + _V)
    scls = jnp.sum(jnp.where(clsm, sp, 0.0))
    xl = jnp.sum(jnp.where(lanes == ll, x, 0.0))
    li = jax.lax.broadcasted_iota(jnp.int32, (1, 128), 1)
    contrib = wgt * (jnp.where(li == 0, spc - xc, 0.0)
                     + jnp.where(li == 1, spc, 0.0)
                     + jnp.where(li == 2, scls - xl, 0.0))

    @pl.when(k == 0)
    def _():
        acc_ref[...] = jnp.zeros_like(acc_ref)

    acc_ref[...] += contrib


def _dense_body(x1_ref, x2_ref, x3_ref, acc_ref):
    i = pl.program_id(0)
    total = jnp.float32(0.0)
    for xr in (x1_ref, x2_ref, x3_ref):
        conf = xr[:, 4:5]
        total += jnp.sum(_softplus(conf))
    li = jax.lax.broadcasted_iota(jnp.int32, (1, 128), 1)

    @pl.when(i == 0)
    def _():
        acc_ref[...] = jnp.zeros_like(acc_ref)

    acc_ref[...] += jnp.where(li == 0, total, 0.0)


def _make_combine_body(M):
    def body(dacc_ref, ga_ref, gb_ref, gc_ref, nobj_ref,
             loss_ref, coord_ref, conf_ref, cls_ref):
        li = jax.lax.broadcasted_iota(jnp.int32, (1, 128), 1)
        gsum = ga_ref[...] + gb_ref[...] + gc_ref[...]
        e1s = jnp.sum(jnp.where(li == 0, gsum, 0.0))
        spcs = jnp.sum(jnp.where(li == 1, gsum, 0.0))
        clss = jnp.sum(jnp.where(li == 2, gsum, 0.0))
        sall = jnp.sum(jnp.where(li == 0, dacc_ref[...], 0.0))
        ri = jax.lax.broadcasted_iota(jnp.int32, (8, 128), 0)
        ci = jax.lax.broadcasted_iota(jnp.int32, (8, 128), 1)
        nobj = jnp.sum(jnp.where((ri == 0) & (ci == 0), nobj_ref[...], 0.0))
        mf = jnp.float32(M)
        conf_loss = e1s / nobj + 0.5 * ((sall - spcs) / (mf - nobj))
        class_loss = 0.5 * (clss / (nobj * jnp.float32(_C)))
        loss_ref[...] = jnp.reshape(conf_loss + class_loss, (1, 1))
        coord_ref[...] = jnp.zeros((1, 1), jnp.float32)
        conf_ref[...] = jnp.reshape(conf_loss, (1, 1))
        cls_ref[...] = jnp.reshape(class_loss, (1, 1))

    return body


def kernel(pred_s1, pred_s2, pred_s3, bboxes, labels, anchors):
    B, N = labels.shape
    preds = [pred_s1, pred_s2, pred_s3]
    grids = tuple(int(p.shape[1]) for p in preds)
    BN = B * N
    M = sum(B * g * g * 3 for g in grids)

    bx = bboxes[..., 0]
    by = bboxes[..., 1]
    bw = bboxes[..., 2]
    bh = bboxes[..., 3]
    aw = anchors[:, 0].reshape(1, 1, -1)
    ah = anchors[:, 1].reshape(1, 1, -1)

    i32 = jnp.int32
    out_shapes = [
        jax.ShapeDtypeStruct((B, N), i32),
        jax.ShapeDtypeStruct((B, N), i32),
        jax.ShapeDtypeStruct((B, N), i32),
        jax.ShapeDtypeStruct((B, N), i32),
        jax.ShapeDtypeStruct((B, N), i32),
        jax.ShapeDtypeStruct((8, 128), jnp.float32),
    ]
    rows1, rows2, rows3, base, labl, nobjv = pl.pallas_call(
        _make_assign_body(B, N, grids), out_shape=out_shapes,
    )(bx, by, bw, bh, labels, aw, ah)

    rows_all = [rows1.reshape(-1), rows2.reshape(-1), rows3.reshape(-1)]
    base_f = base.reshape(-1)
    labl_f = labl.reshape(-1)

    gaccs = []
    for p, g, rows in zip(preds, grids, rows_all):
        view = p.reshape(B * g * g, 1, 3 * _V)
        grid_spec = pltpu.PrefetchScalarGridSpec(
            num_scalar_prefetch=3,
            grid=(BN,),
            in_specs=[pl.BlockSpec(
                (1, 1, 3 * _V),
                lambda k, r, bb, ll: (jnp.maximum(r[k], 0), 0, 0))],
            out_specs=pl.BlockSpec((1, 128), lambda k, r, bb, ll: (0, 0)),
        )
        acc = pl.pallas_call(
            _gather_body, grid_spec=grid_spec,
            out_shape=jax.ShapeDtypeStruct((1, 128), jnp.float32),
        )(rows, base_f, labl_f, view)
        gaccs.append(acc)

    steps = 169
    views = [p.reshape(B * g * g * 3, _V) for p, g in zip(preds, grids)]
    blocks = [v.shape[0] // steps for v in views]
    dense_spec = [
        pl.BlockSpec((blk, _V), lambda i: (i, 0)) for blk in blocks
    ]
    dacc = pl.pallas_call(
        _dense_body,
        grid=(steps,),
        in_specs=dense_spec,
        out_specs=pl.BlockSpec((1, 128), lambda i: (0, 0)),
        out_shape=jax.ShapeDtypeStruct((1, 128), jnp.float32),
    )(*views)

    outs = pl.pallas_call(
        _make_combine_body(M),
        out_shape=[jax.ShapeDtypeStruct((1, 1), jnp.float32)] * 4,
    )(dacc, gaccs[0], gaccs[1], gaccs[2], nobjv)
    loss, coord, conf, cls = [o.reshape(()) for o in outs]
    return (loss, coord, conf, cls)


# speedup vs baseline: 31.8475x; 2.4229x over previous
import jax
import jax.numpy as jnp
from jax.experimental import pallas as pl
from jax.experimental.pallas import tpu as pltpu

_C = 80
_V = 5 + _C


def _two_sum(a, b):
    s = a + b
    bv = s - a
    e = (a - (s - bv)) + (b - bv)
    return s, e


def _floor_scaled(xs, xe, gf):
    p = xs * gf
    c = jnp.float32(4097.0) * xs
    xhi = c - (c - xs)
    xlo = xs - xhi
    err = (xhi * gf - p) + xlo * gf
    lo = err + xe * gf
    c0 = jnp.floor(p)
    r = p - c0
    adj = jnp.where(lo >= (jnp.float32(1.0) - r), 1.0,
                    jnp.where(lo < -r, -1.0, 0.0))
    return (c0 + adj).astype(jnp.int32)


def _softplus(x):
    return jnp.maximum(x, 0.0) + jnp.log1p(jnp.exp(-jnp.abs(x)))


def _make_assign_body(B, N, grids):
    g1, g2, g3 = grids
    c1 = g1 * g1 * 3
    c2 = g2 * g2 * 3

    def body(bx_ref, by_ref, bw_ref, bh_ref, lab_ref, aw_ref, ah_ref,
             rows1_ref, rows2_ref, rows3_ref, base_ref, labl_ref, nobj_ref):
        x = bx_ref[...]
        w = bw_ref[...]
        y = by_ref[...]
        h = bh_ref[...]
        cxs, cxe = _two_sum(x, w * 0.5)
        cys, cye = _two_sum(y, h * 0.5)
        aw = aw_ref[...]
        ah = ah_ref[...]
        d1, e1 = _two_sum(w[:, :, None], -aw)
        d2, e2 = _two_sum(h[:, :, None], -ah)
        a1 = jnp.abs(d1)
        ae1 = jnp.where(d1 < 0, -e1, e1)
        a2 = jnp.abs(d2)
        ae2 = jnp.where(d2 < 0, -e2, e2)
        hi0, er = _two_sum(a1, a2)
        lo0 = er + ae1 + ae2
        t = hi0 + lo0
        lo = lo0 - (t - hi0)
        hi = t
        hm = jnp.min(hi, axis=-1, keepdims=True)
        m1 = hi == hm
        lm = jnp.min(jnp.where(m1, lo, jnp.float32(jnp.inf)), axis=-1,
                     keepdims=True)
        m2 = m1 & (lo == lm)
        j9 = jax.lax.broadcasted_iota(jnp.int32, m2.shape, 2)
        prior = jnp.min(jnp.where(m2, j9, 9), axis=-1)
        tidx = prior // 3
        aidx = prior - tidx * 3
        g = jnp.where(tidx == 0, g1, jnp.where(tidx == 1, g2, g3))
        gf = g.astype(jnp.float32)
        cxi = _floor_scaled(cxs, cxe, gf)
        cyi = _floor_scaled(cys, cye, gf)
        valid = (cyi >= 0) & (cyi < g) & (cxi >= 0) & (cxi < g)
        offs = jnp.where(tidx == 0, 0, jnp.where(tidx == 1, c1, c1 + c2))
        key = offs + (cyi * g + cxi) * 3 + aidx
        eq = key[:, :, None] == key[:, None, :]
        vm = valid[:, None, :]
        nn = jax.lax.broadcasted_iota(jnp.int32, (B, N, N), 1)
        mm = jax.lax.broadcasted_iota(jnp.int32, (B, N, N), 2)
        clob = jnp.sum((eq & vm & (mm > nn)).astype(jnp.int32), axis=2) > 0
        winner = valid & jnp.logical_not(clob)
        b2 = jax.lax.broadcasted_iota(jnp.int32, (B, N), 0)
        for s, gv, rref in ((0, g1, rows1_ref), (1, g2, rows2_ref),
                            (2, g3, rows3_ref)):
            sel = winner & (tidx == s)
            row = (b2 * gv + cyi) * gv + cxi
            rref[...] = jnp.where(sel, row, -1)
        base_ref[...] = aidx * _V
        labl_ref[...] = aidx * _V + 5 + lab_ref[...]
        nobj = jnp.sum(winner.astype(jnp.float32))
        ri = jax.lax.broadcasted_iota(jnp.int32, (8, 128), 0)
        ci = jax.lax.broadcasted_iota(jnp.int32, (8, 128), 1)
        nobj_ref[...] = jnp.where((ri == 0) & (ci == 0), nobj, 0.0)

    return body


_GWAYS = 8


def _gather_body(rows_ref, base_ref, labl_ref, *refs):
    x_refs = refs[:_GWAYS]
    acc_ref = refs[_GWAYS]
    k = pl.program_id(0)
    li = jax.lax.broadcasted_iota(jnp.int32, (1, 128), 1)
    contrib = jnp.zeros((1, 128), jnp.float32)
    for j in range(_GWAYS):
        idx = k * _GWAYS + j
        row = rows_ref[idx]
        wgt = jnp.where(row >= 0, jnp.float32(1.0), jnp.float32(0.0))
        sub = row - (row // 8) * 8
        b = base_ref[idx]
        ll = labl_ref[idx]
        x = x_refs[j][...]
        lanes = jax.lax.broadcasted_iota(jnp.int32, x.shape, 1)
        subs = jax.lax.broadcasted_iota(jnp.int32, x.shape, 0)
        onrow = subs == sub
        sp = _softplus(x)
        isconf = onrow & (lanes == b + 4)
        spc = jnp.sum(jnp.where(isconf, sp, 0.0))
        xc = jnp.sum(jnp.where(isconf, x, 0.0))
        clsm = onrow & (lanes >= b + 5) & (lanes < b + _V)
        scls = jnp.sum(jnp.where(clsm, sp, 0.0))
        xl = jnp.sum(jnp.where(onrow & (lanes == ll), x, 0.0))
        contrib += wgt * (jnp.where(li == 0, spc - xc, 0.0)
                          + jnp.where(li == 1, spc, 0.0)
                          + jnp.where(li == 2, scls - xl, 0.0))

    @pl.when(k == 0)
    def _():
        acc_ref[...] = jnp.zeros_like(acc_ref)

    acc_ref[...] += contrib


def _dense_body(x1_ref, x2_ref, x3_ref, acc_ref):
    i = pl.program_id(0)
    total = jnp.float32(0.0)
    for xr in (x1_ref, x2_ref, x3_ref):
        x = xr[...]
        for lane in (4, 89, 174):
            total += jnp.sum(_softplus(x[:, lane:lane + 1]))
    li = jax.lax.broadcasted_iota(jnp.int32, (1, 128), 1)

    @pl.when(i == 0)
    def _():
        acc_ref[...] = jnp.zeros_like(acc_ref)

    acc_ref[...] += jnp.where(li == 0, total, 0.0)


def _make_combine_body(M):
    def body(dacc_ref, ga_ref, gb_ref, gc_ref, nobj_ref,
             loss_ref, coord_ref, conf_ref, cls_ref):
        li = jax.lax.broadcasted_iota(jnp.int32, (1, 128), 1)
        gsum = ga_ref[...] + gb_ref[...] + gc_ref[...]
        e1s = jnp.sum(jnp.where(li == 0, gsum, 0.0))
        spcs = jnp.sum(jnp.where(li == 1, gsum, 0.0))
        clss = jnp.sum(jnp.where(li == 2, gsum, 0.0))
        sall = jnp.sum(jnp.where(li == 0, dacc_ref[...], 0.0))
        ri = jax.lax.broadcasted_iota(jnp.int32, (8, 128), 0)
        ci = jax.lax.broadcasted_iota(jnp.int32, (8, 128), 1)
        nobj = jnp.sum(jnp.where((ri == 0) & (ci == 0), nobj_ref[...], 0.0))
        mf = jnp.float32(M)
        conf_loss = e1s / nobj + 0.5 * ((sall - spcs) / (mf - nobj))
        class_loss = 0.5 * (clss / (nobj * jnp.float32(_C)))
        loss_ref[...] = jnp.reshape(conf_loss + class_loss, (1, 1))
        coord_ref[...] = jnp.zeros((1, 1), jnp.float32)
        conf_ref[...] = jnp.reshape(conf_loss, (1, 1))
        cls_ref[...] = jnp.reshape(class_loss, (1, 1))

    return body


def kernel(pred_s1, pred_s2, pred_s3, bboxes, labels, anchors):
    B, N = labels.shape
    preds = [pred_s1, pred_s2, pred_s3]
    grids = tuple(int(p.shape[1]) for p in preds)
    BN = B * N
    M = sum(B * g * g * 3 for g in grids)

    bx = bboxes[..., 0]
    by = bboxes[..., 1]
    bw = bboxes[..., 2]
    bh = bboxes[..., 3]
    aw = anchors[:, 0].reshape(1, 1, -1)
    ah = anchors[:, 1].reshape(1, 1, -1)

    i32 = jnp.int32
    out_shapes = [
        jax.ShapeDtypeStruct((B, N), i32),
        jax.ShapeDtypeStruct((B, N), i32),
        jax.ShapeDtypeStruct((B, N), i32),
        jax.ShapeDtypeStruct((B, N), i32),
        jax.ShapeDtypeStruct((B, N), i32),
        jax.ShapeDtypeStruct((8, 128), jnp.float32),
    ]
    rows1, rows2, rows3, base, labl, nobjv = pl.pallas_call(
        _make_assign_body(B, N, grids), out_shape=out_shapes,
    )(bx, by, bw, bh, labels, aw, ah)

    rows_all = [rows1.reshape(-1), rows2.reshape(-1), rows3.reshape(-1)]
    base_f = base.reshape(-1)
    labl_f = labl.reshape(-1)

    views = [p.reshape(B * g * g, 3 * _V) for p, g in zip(preds, grids)]

    gaccs = []
    for view, rows in zip(views, rows_all):
        def _mk_map(j):
            def im(k, r, bb, ll):
                return (jnp.maximum(r[k * _GWAYS + j], 0) // 8, 0)
            return im

        grid_spec = pltpu.PrefetchScalarGridSpec(
            num_scalar_prefetch=3,
            grid=(BN // _GWAYS,),
            in_specs=[pl.BlockSpec((8, 3 * _V), _mk_map(j))
                      for j in range(_GWAYS)],
            out_specs=pl.BlockSpec((1, 128), lambda k, r, bb, ll: (0, 0)),
        )
        acc = pl.pallas_call(
            _gather_body, grid_spec=grid_spec,
            out_shape=jax.ShapeDtypeStruct((1, 128), jnp.float32),
        )(rows, base_f, labl_f, *([view] * _GWAYS))
        gaccs.append(acc)

    steps = 169
    dense_spec = [
        pl.BlockSpec((v.shape[0] // steps, 3 * _V), lambda i: (i, 0))
        for v in views
    ]
    dacc = pl.pallas_call(
        _dense_body,
        grid=(steps,),
        in_specs=dense_spec,
        out_specs=pl.BlockSpec((1, 128), lambda i: (0, 0)),
        out_shape=jax.ShapeDtypeStruct((1, 128), jnp.float32),
    )(*views)

    outs = pl.pallas_call(
        _make_combine_body(M),
        out_shape=[jax.ShapeDtypeStruct((1, 1), jnp.float32)] * 4,
    )(dacc, gaccs[0], gaccs[1], gaccs[2], nobjv)
    loss, coord, conf, cls = [o.reshape(()) for o in outs]
    return (loss, coord, conf, cls)


# speedup vs baseline: 42.7821x; 1.3433x over previous
import jax
import jax.numpy as jnp
from jax.experimental import pallas as pl
from jax.experimental.pallas import tpu as pltpu

_C = 80
_V = 5 + _C


def _two_sum(a, b):
    s = a + b
    bv = s - a
    e = (a - (s - bv)) + (b - bv)
    return s, e


def _floor_scaled(xs, xe, gf):
    p = xs * gf
    c = jnp.float32(4097.0) * xs
    xhi = c - (c - xs)
    xlo = xs - xhi
    err = (xhi * gf - p) + xlo * gf
    lo = err + xe * gf
    c0 = jnp.floor(p)
    r = p - c0
    adj = jnp.where(lo >= (jnp.float32(1.0) - r), 1.0,
                    jnp.where(lo < -r, -1.0, 0.0))
    return (c0 + adj).astype(jnp.int32)


def _softplus(x):
    return jnp.maximum(x, 0.0) + jnp.log1p(jnp.exp(-jnp.abs(x)))


def _make_assign_body(B, N, grids):
    g1, g2, g3 = grids
    c1 = g1 * g1 * 3
    c2 = g2 * g2 * 3

    def body(bx_ref, by_ref, bw_ref, bh_ref, lab_ref, aw_ref, ah_ref,
             rows1_ref, rows2_ref, rows3_ref, base_ref, labl_ref, nobj_ref):
        x = bx_ref[...]
        w = bw_ref[...]
        y = by_ref[...]
        h = bh_ref[...]
        cxs, cxe = _two_sum(x, w * 0.5)
        cys, cye = _two_sum(y, h * 0.5)
        aw = aw_ref[...]
        ah = ah_ref[...]
        d1, e1 = _two_sum(w[:, :, None], -aw)
        d2, e2 = _two_sum(h[:, :, None], -ah)
        a1 = jnp.abs(d1)
        ae1 = jnp.where(d1 < 0, -e1, e1)
        a2 = jnp.abs(d2)
        ae2 = jnp.where(d2 < 0, -e2, e2)
        hi0, er = _two_sum(a1, a2)
        lo0 = er + ae1 + ae2
        t = hi0 + lo0
        lo = lo0 - (t - hi0)
        hi = t
        hm = jnp.min(hi, axis=-1, keepdims=True)
        m1 = hi == hm
        lm = jnp.min(jnp.where(m1, lo, jnp.float32(jnp.inf)), axis=-1,
                     keepdims=True)
        m2 = m1 & (lo == lm)
        j9 = jax.lax.broadcasted_iota(jnp.int32, m2.shape, 2)
        prior = jnp.min(jnp.where(m2, j9, 9), axis=-1)
        tidx = prior // 3
        aidx = prior - tidx * 3
        g = jnp.where(tidx == 0, g1, jnp.where(tidx == 1, g2, g3))
        gf = g.astype(jnp.float32)
        cxi = _floor_scaled(cxs, cxe, gf)
        cyi = _floor_scaled(cys, cye, gf)
        valid = (cyi >= 0) & (cyi < g) & (cxi >= 0) & (cxi < g)
        offs = jnp.where(tidx == 0, 0, jnp.where(tidx == 1, c1, c1 + c2))
        key = offs + (cyi * g + cxi) * 3 + aidx
        eq = key[:, :, None] == key[:, None, :]
        vm = valid[:, None, :]
        nn = jax.lax.broadcasted_iota(jnp.int32, (B, N, N), 1)
        mm = jax.lax.broadcasted_iota(jnp.int32, (B, N, N), 2)
        clob = jnp.sum((eq & vm & (mm > nn)).astype(jnp.int32), axis=2) > 0
        winner = valid & jnp.logical_not(clob)
        b2 = jax.lax.broadcasted_iota(jnp.int32, (B, N), 0)
        for s, gv, rref in ((0, g1, rows1_ref), (1, g2, rows2_ref),
                            (2, g3, rows3_ref)):
            sel = winner & (tidx == s)
            row = (b2 * gv + cyi) * gv + cxi
            rref[...] = jnp.where(sel, row, -1)
        base_ref[...] = aidx * _V
        labl_ref[...] = aidx * _V + 5 + lab_ref[...]
        nobj = jnp.sum(winner.astype(jnp.float32))
        ri = jax.lax.broadcasted_iota(jnp.int32, (8, 128), 0)
        ci = jax.lax.broadcasted_iota(jnp.int32, (8, 128), 1)
        nobj_ref[...] = jnp.where((ri == 0) & (ci == 0), nobj, 0.0)

    return body


_GWAYS = 8


def _make_gather_body(g):
    def body(rows_ref, base_ref, labl_ref, *refs):
        x_refs = refs[:_GWAYS]
        acc_ref = refs[_GWAYS]
        k = pl.program_id(0)
        li = jax.lax.broadcasted_iota(jnp.int32, (1, 128), 1)
        contrib = jnp.zeros((1, 128), jnp.float32)
        for j in range(_GWAYS):
            idx = k * _GWAYS + j
            row = rows_ref[idx]
            wgt = jnp.where(row >= 0, jnp.float32(1.0), jnp.float32(0.0))
            cx = row - (row // g) * g
            b = base_ref[idx]
            ll = labl_ref[idx]
            x = x_refs[j][0, 0]
            lanes = jax.lax.broadcasted_iota(jnp.int32, x.shape, 1)
            subs = jax.lax.broadcasted_iota(jnp.int32, x.shape, 0)
            onrow = subs == cx
            sp = _softplus(x)
            isconf = onrow & (lanes == b + 4)
            spc = jnp.sum(jnp.where(isconf, sp, 0.0))
            xc = jnp.sum(jnp.where(isconf, x, 0.0))
            clsm = onrow & (lanes >= b + 5) & (lanes < b + _V)
            scls = jnp.sum(jnp.where(clsm, sp, 0.0))
            xl = jnp.sum(jnp.where(onrow & (lanes == ll), x, 0.0))
            contrib += wgt * (jnp.where(li == 0, spc - xc, 0.0)
                              + jnp.where(li == 1, spc, 0.0)
                              + jnp.where(li == 2, scls - xl, 0.0))

        @pl.when(k == 0)
        def _():
            acc_ref[...] = jnp.zeros_like(acc_ref)

        acc_ref[...] += contrib

    return body


def _dense_body(x1_ref, x2_ref, x3_ref, acc_ref):
    i = pl.program_id(0)
    total = jnp.float32(0.0)
    for xr in (x1_ref, x2_ref, x3_ref):
        x = xr[0]
        for lane in (4, 89, 174):
            total += jnp.sum(_softplus(x[:, :, lane:lane + 1]))
    li = jax.lax.broadcasted_iota(jnp.int32, (1, 128), 1)

    @pl.when(i == 0)
    def _():
        acc_ref[...] = jnp.zeros_like(acc_ref)

    acc_ref[...] += jnp.where(li == 0, total, 0.0)


def _make_combine_body(M):
    def body(dacc_ref, ga_ref, gb_ref, gc_ref, nobj_ref,
             loss_ref, coord_ref, conf_ref, cls_ref):
        li = jax.lax.broadcasted_iota(jnp.int32, (1, 128), 1)
        gsum = ga_ref[...] + gb_ref[...] + gc_ref[...]
        e1s = jnp.sum(jnp.where(li == 0, gsum, 0.0))
        spcs = jnp.sum(jnp.where(li == 1, gsum, 0.0))
        clss = jnp.sum(jnp.where(li == 2, gsum, 0.0))
        sall = jnp.sum(jnp.where(li == 0, dacc_ref[...], 0.0))
        ri = jax.lax.broadcasted_iota(jnp.int32, (8, 128), 0)
        ci = jax.lax.broadcasted_iota(jnp.int32, (8, 128), 1)
        nobj = jnp.sum(jnp.where((ri == 0) & (ci == 0), nobj_ref[...], 0.0))
        mf = jnp.float32(M)
        conf_loss = e1s / nobj + 0.5 * ((sall - spcs) / (mf - nobj))
        class_loss = 0.5 * (clss / (nobj * jnp.float32(_C)))
        loss_ref[...] = jnp.reshape(conf_loss + class_loss, (1, 1))
        coord_ref[...] = jnp.zeros((1, 1), jnp.float32)
        conf_ref[...] = jnp.reshape(conf_loss, (1, 1))
        cls_ref[...] = jnp.reshape(class_loss, (1, 1))

    return body


def kernel(pred_s1, pred_s2, pred_s3, bboxes, labels, anchors):
    B, N = labels.shape
    preds = [pred_s1, pred_s2, pred_s3]
    grids = tuple(int(p.shape[1]) for p in preds)
    BN = B * N
    M = sum(B * g * g * 3 for g in grids)

    bx = bboxes[..., 0]
    by = bboxes[..., 1]
    bw = bboxes[..., 2]
    bh = bboxes[..., 3]
    aw = anchors[:, 0].reshape(1, 1, -1)
    ah = anchors[:, 1].reshape(1, 1, -1)

    i32 = jnp.int32
    out_shapes = [
        jax.ShapeDtypeStruct((B, N), i32),
        jax.ShapeDtypeStruct((B, N), i32),
        jax.ShapeDtypeStruct((B, N), i32),
        jax.ShapeDtypeStruct((B, N), i32),
        jax.ShapeDtypeStruct((B, N), i32),
        jax.ShapeDtypeStruct((8, 128), jnp.float32),
    ]
    rows1, rows2, rows3, base, labl, nobjv = pl.pallas_call(
        _make_assign_body(B, N, grids), out_shape=out_shapes,
    )(bx, by, bw, bh, labels, aw, ah)

    rows_all = [rows1.reshape(-1), rows2.reshape(-1), rows3.reshape(-1)]
    base_f = base.reshape(-1)
    labl_f = labl.reshape(-1)

    gaccs = []
    for p, g, rows in zip(preds, grids, rows_all):
        def _mk_map(j, gv):
            def im(k, r, bb, ll):
                row = jnp.maximum(r[k * _GWAYS + j], 0)
                b = row // (gv * gv)
                cy = (row - b * gv * gv) // gv
                return (b, cy, 0, 0)
            return im

        grid_spec = pltpu.PrefetchScalarGridSpec(
            num_scalar_prefetch=3,
            grid=(BN // _GWAYS,),
            in_specs=[pl.BlockSpec((1, 1, g, 3 * _V), _mk_map(j, g))
                      for j in range(_GWAYS)],
            out_specs=pl.BlockSpec((1, 128), lambda k, r, bb, ll: (0, 0)),
        )
        acc = pl.pallas_call(
            _make_gather_body(g), grid_spec=grid_spec,
            out_shape=jax.ShapeDtypeStruct((1, 128), jnp.float32),
        )(rows, base_f, labl_f, *([p] * _GWAYS))
        gaccs.append(acc)

    dense_spec = [
        pl.BlockSpec((1, g, g, 3 * _V), lambda i: (i, 0, 0, 0))
        for g in grids
    ]
    dacc = pl.pallas_call(
        _dense_body,
        grid=(B,),
        in_specs=dense_spec,
        out_specs=pl.BlockSpec((1, 128), lambda i: (0, 0)),
        out_shape=jax.ShapeDtypeStruct((1, 128), jnp.float32),
    )(*preds)

    outs = pl.pallas_call(
        _make_combine_body(M),
        out_shape=[jax.ShapeDtypeStruct((1, 1), jnp.float32)] * 4,
    )(dacc, gaccs[0], gaccs[1], gaccs[2], nobjv)
    loss, coord, conf, cls = [o.reshape(()) for o in outs]
    return (loss, coord, conf, cls)


# speedup vs baseline: 47.3983x; 1.1079x over previous
import jax
import jax.numpy as jnp
from jax.experimental import pallas as pl
from jax.experimental.pallas import tpu as pltpu

_C = 80
_V = 5 + _C


def _two_sum(a, b):
    s = a + b
    bv = s - a
    e = (a - (s - bv)) + (b - bv)
    return s, e


def _floor_scaled(xs, xe, gf):
    p = xs * gf
    c = jnp.float32(4097.0) * xs
    xhi = c - (c - xs)
    xlo = xs - xhi
    err = (xhi * gf - p) + xlo * gf
    lo = err + xe * gf
    c0 = jnp.floor(p)
    r = p - c0
    adj = jnp.where(lo >= (jnp.float32(1.0) - r), 1.0,
                    jnp.where(lo < -r, -1.0, 0.0))
    return (c0 + adj).astype(jnp.int32)


def _softplus(x):
    return jnp.maximum(x, 0.0) + jnp.log1p(jnp.exp(-jnp.abs(x)))


def _make_assign_body(B, N, grids):
    g1, g2, g3 = grids
    c1 = g1 * g1 * 3
    c2 = g2 * g2 * 3

    def body(bx_ref, by_ref, bw_ref, bh_ref, lab_ref, aw_ref, ah_ref,
             rows1_ref, rows2_ref, rows3_ref, base_ref, labl_ref, nobj_ref):
        x = bx_ref[...]
        w = bw_ref[...]
        y = by_ref[...]
        h = bh_ref[...]
        cxs, cxe = _two_sum(x, w * 0.5)
        cys, cye = _two_sum(y, h * 0.5)
        aw = aw_ref[...]
        ah = ah_ref[...]
        d1, e1 = _two_sum(w[:, :, None], -aw)
        d2, e2 = _two_sum(h[:, :, None], -ah)
        a1 = jnp.abs(d1)
        ae1 = jnp.where(d1 < 0, -e1, e1)
        a2 = jnp.abs(d2)
        ae2 = jnp.where(d2 < 0, -e2, e2)
        hi0, er = _two_sum(a1, a2)
        lo0 = er + ae1 + ae2
        t = hi0 + lo0
        lo = lo0 - (t - hi0)
        hi = t
        hm = jnp.min(hi, axis=-1, keepdims=True)
        m1 = hi == hm
        lm = jnp.min(jnp.where(m1, lo, jnp.float32(jnp.inf)), axis=-1,
                     keepdims=True)
        m2 = m1 & (lo == lm)
        j9 = jax.lax.broadcasted_iota(jnp.int32, m2.shape, 2)
        prior = jnp.min(jnp.where(m2, j9, 9), axis=-1)
        tidx = prior // 3
        aidx = prior - tidx * 3
        g = jnp.where(tidx == 0, g1, jnp.where(tidx == 1, g2, g3))
        gf = g.astype(jnp.float32)
        cxi = _floor_scaled(cxs, cxe, gf)
        cyi = _floor_scaled(cys, cye, gf)
        valid = (cyi >= 0) & (cyi < g) & (cxi >= 0) & (cxi < g)
        offs = jnp.where(tidx == 0, 0, jnp.where(tidx == 1, c1, c1 + c2))
        key = offs + (cyi * g + cxi) * 3 + aidx
        eq = key[:, :, None] == key[:, None, :]
        vm = valid[:, None, :]
        nn = jax.lax.broadcasted_iota(jnp.int32, (B, N, N), 1)
        mm = jax.lax.broadcasted_iota(jnp.int32, (B, N, N), 2)
        clob = jnp.sum((eq & vm & (mm > nn)).astype(jnp.int32), axis=2) > 0
        winner = valid & jnp.logical_not(clob)
        b2 = jax.lax.broadcasted_iota(jnp.int32, (B, N), 0)
        for s, gv, rref in ((0, g1, rows1_ref), (1, g2, rows2_ref),
                            (2, g3, rows3_ref)):
            sel = winner & (tidx == s)
            row = (b2 * gv + cyi) * gv + cxi
            rref[...] = jnp.where(sel, row, -1)
        base_ref[...] = aidx * _V
        labl_ref[...] = aidx * _V + 5 + lab_ref[...]
        nobj = jnp.sum(winner.astype(jnp.float32))
        ri = jax.lax.broadcasted_iota(jnp.int32, (8, 128), 0)
        ci = jax.lax.broadcasted_iota(jnp.int32, (8, 128), 1)
        nobj_ref[...] = jnp.where((ri == 0) & (ci == 0), nobj, 0.0)

    return body


_GWAYS = 8


def _make_gather_body(grids):
    def body(r1_ref, r2_ref, r3_ref, base_ref, labl_ref, *refs):
        x_refs = refs[:3 * _GWAYS]
        acc_ref = refs[3 * _GWAYS]
        k = pl.program_id(0)
        li = jax.lax.broadcasted_iota(jnp.int32, (1, 128), 1)
        contrib = jnp.zeros((1, 128), jnp.float32)
        for s, (g, rows_ref) in enumerate(zip(grids,
                                              (r1_ref, r2_ref, r3_ref))):
            for j in range(_GWAYS):
                idx = k * _GWAYS + j
                row = rows_ref[idx]
                wgt = jnp.where(row >= 0, jnp.float32(1.0), jnp.float32(0.0))
                cx = row - (row // g) * g
                b = base_ref[idx]
                ll = labl_ref[idx]
                x = x_refs[s * _GWAYS + j][0, 0]
                lanes = jax.lax.broadcasted_iota(jnp.int32, x.shape, 1)
                subs = jax.lax.broadcasted_iota(jnp.int32, x.shape, 0)
                onrow = subs == cx
                sp = _softplus(x)
                isconf = onrow & (lanes == b + 4)
                spc = jnp.sum(jnp.where(isconf, sp, 0.0))
                xc = jnp.sum(jnp.where(isconf, x, 0.0))
                clsm = onrow & (lanes >= b + 5) & (lanes < b + _V)
                scls = jnp.sum(jnp.where(clsm, sp, 0.0))
                xl = jnp.sum(jnp.where(onrow & (lanes == ll), x, 0.0))
                contrib += wgt * (jnp.where(li == 0, spc - xc, 0.0)
                                  + jnp.where(li == 1, spc, 0.0)
                                  + jnp.where(li == 2, scls - xl, 0.0))

        @pl.when(k == 0)
        def _():
            acc_ref[...] = jnp.zeros_like(acc_ref)

        acc_ref[...] += contrib

    return body


def _make_dense_combine_body(M, B):
    def body(x1_ref, x2_ref, x3_ref, gacc_ref, nobj_ref,
             loss_ref, coord_ref, conf_ref, cls_ref, acc_ref):
        i = pl.program_id(0)
        total = jnp.float32(0.0)
        for xr in (x1_ref, x2_ref, x3_ref):
            x = xr[0]
            for lane in (4, 89, 174):
                total += jnp.sum(_softplus(x[:, :, lane:lane + 1]))
        li = jax.lax.broadcasted_iota(jnp.int32, (1, 128), 1)

        @pl.when(i == 0)
        def _():
            acc_ref[...] = jnp.zeros_like(acc_ref)

        acc_ref[...] += jnp.where(li == 0, total, 0.0)

        @pl.when(i == B - 1)
        def _():
            gsum = gacc_ref[...]
            e1s = jnp.sum(jnp.where(li == 0, gsum, 0.0))
            spcs = jnp.sum(jnp.where(li == 1, gsum, 0.0))
            clss = jnp.sum(jnp.where(li == 2, gsum, 0.0))
            sall = jnp.sum(jnp.where(li == 0, acc_ref[...], 0.0))
            ri = jax.lax.broadcasted_iota(jnp.int32, (8, 128), 0)
            ci = jax.lax.broadcasted_iota(jnp.int32, (8, 128), 1)
            nobj = jnp.sum(jnp.where((ri == 0) & (ci == 0),
                                     nobj_ref[...], 0.0))
            mf = jnp.float32(M)
            conf_loss = e1s / nobj + 0.5 * ((sall - spcs) / (mf - nobj))
            class_loss = 0.5 * (clss / (nobj * jnp.float32(_C)))
            loss_ref[...] = jnp.reshape(conf_loss + class_loss, (1, 1))
            coord_ref[...] = jnp.zeros((1, 1), jnp.float32)
            conf_ref[...] = jnp.reshape(conf_loss, (1, 1))
            cls_ref[...] = jnp.reshape(class_loss, (1, 1))

    return body


def kernel(pred_s1, pred_s2, pred_s3, bboxes, labels, anchors):
    B, N = labels.shape
    preds = [pred_s1, pred_s2, pred_s3]
    grids = tuple(int(p.shape[1]) for p in preds)
    BN = B * N
    M = sum(B * g * g * 3 for g in grids)

    bx = bboxes[..., 0]
    by = bboxes[..., 1]
    bw = bboxes[..., 2]
    bh = bboxes[..., 3]
    aw = anchors[:, 0].reshape(1, 1, -1)
    ah = anchors[:, 1].reshape(1, 1, -1)

    i32 = jnp.int32
    out_shapes = [
        jax.ShapeDtypeStruct((B, N), i32),
        jax.ShapeDtypeStruct((B, N), i32),
        jax.ShapeDtypeStruct((B, N), i32),
        jax.ShapeDtypeStruct((B, N), i32),
        jax.ShapeDtypeStruct((B, N), i32),
        jax.ShapeDtypeStruct((8, 128), jnp.float32),
    ]
    rows1, rows2, rows3, base, labl, nobjv = pl.pallas_call(
        _make_assign_body(B, N, grids), out_shape=out_shapes,
    )(bx, by, bw, bh, labels, aw, ah)

    rows_all = [rows1.reshape(-1), rows2.reshape(-1), rows3.reshape(-1)]
    base_f = base.reshape(-1)
    labl_f = labl.reshape(-1)

    def _mk_map(s, j, gv):
        def im(k, r1, r2, r3, bb, ll):
            r = (r1, r2, r3)[s]
            row = jnp.maximum(r[k * _GWAYS + j], 0)
            b = row // (gv * gv)
            cy = (row - b * gv * gv) // gv
            return (b, cy, 0, 0)
        return im

    gather_specs = [
        pl.BlockSpec((1, 1, g, 3 * _V), _mk_map(s, j, g))
        for s, g in enumerate(grids) for j in range(_GWAYS)
    ]
    gather_inputs = [p for p in preds for _ in range(_GWAYS)]
    grid_spec = pltpu.PrefetchScalarGridSpec(
        num_scalar_prefetch=5,
        grid=(BN // _GWAYS,),
        in_specs=gather_specs,
        out_specs=pl.BlockSpec((1, 128),
                               lambda k, r1, r2, r3, bb, ll: (0, 0)),
    )
    gacc = pl.pallas_call(
        _make_gather_body(grids), grid_spec=grid_spec,
        out_shape=jax.ShapeDtypeStruct((1, 128), jnp.float32),
    )(rows_all[0], rows_all[1], rows_all[2], base_f, labl_f,
      *gather_inputs)

    dense_spec = [
        pl.BlockSpec((1, g, g, 3 * _V), lambda i: (i, 0, 0, 0))
        for g in grids
    ] + [
        pl.BlockSpec((1, 128), lambda i: (0, 0)),
        pl.BlockSpec((8, 128), lambda i: (0, 0)),
    ]
    outs = pl.pallas_call(
        _make_dense_combine_body(M, B),
        grid=(B,),
        in_specs=dense_spec,
        out_specs=[pl.BlockSpec((1, 1), lambda i: (0, 0))] * 4,
        out_shape=[jax.ShapeDtypeStruct((1, 1), jnp.float32)] * 4,
        scratch_shapes=[pltpu.VMEM((1, 128), jnp.float32)],
    )(*preds, gacc, nobjv)
    loss, coord, conf, cls = [o.reshape(()) for o in outs]
    return (loss, coord, conf, cls)
